# baseline (speedup 1.0000x reference)
import functools

import jax
import jax.numpy as jnp
from jax import lax
from jax.experimental import pallas as pl
from jax.experimental.pallas import tpu as pltpu
from jax.experimental.pallas import tpu_sc as plsc

D = 2048
B = 1024
L = 50
LP = 56
NBAGS = 2 * B
NW = 32
BAGS_PER_W = NBAGS // NW
NLANE = 16
CHUNKS = (16, 16, 16, 2)
STARTS = (0, 16, 32, 48)


def _sc_pool(idx_pad, embedding):
    mesh = plsc.VectorSubcoreMesh(core_axis_name="c", subcore_axis_name="s")

    @functools.partial(
        pl.kernel,
        out_type=jax.ShapeDtypeStruct((NBAGS, D), jnp.float32),
        mesh=mesh,
        scratch_types=[
            pltpu.VMEM((BAGS_PER_W, LP), jnp.int32),
            pltpu.VMEM((16, D), jnp.float32),
            pltpu.VMEM((16, D), jnp.float32),
            pltpu.VMEM((16, D), jnp.float32),
            pltpu.VMEM((2, D), jnp.float32),
            pltpu.VMEM((2, D), jnp.float32),
            pltpu.SemaphoreType.DMA,
            pltpu.SemaphoreType.DMA,
            pltpu.SemaphoreType.DMA,
            pltpu.SemaphoreType.DMA,
            pltpu.SemaphoreType.DMA,
        ],
    )
    def pool(idx_hbm, table_hbm, out_hbm, idx_v, buf0, buf1, buf2, buf3,
             acc_v, gsem0, gsem1, gsem2, gsem3, osem):
        nc = plsc.get_sparse_core_info().num_cores
        wid = lax.axis_index("s") * nc + lax.axis_index("c")
        base = wid * BAGS_PER_W
        pltpu.sync_copy(idx_hbm.at[pl.ds(base, BAGS_PER_W)], idx_v)

        bufs = (buf0, buf1, buf2, buf3)
        gsems = (gsem0, gsem1, gsem2, gsem3)
        BUFMAP = (0, 1, 2, 3)

        def fire(bag, j):
            n = CHUNKS[j]
            m = BUFMAP[j]
            pltpu.async_copy(
                table_hbm.at[idx_v.at[bag, pl.ds(STARTS[j], n)]],
                bufs[m],
                gsems[m],
            )

        for j in range(len(CHUNKS)):
            fire(0, j)

        def bag_body(b, carry):
            arow = b & 1
            for j in range(len(CHUNKS)):
                n = CHUNKS[j]
                m = BUFMAP[j]
                if j == 0:
                    @pl.when(b >= 2)
                    def _():
                        pltpu.make_async_copy(
                            acc_v.at[0], out_hbm.at[base], osem).wait()
                pltpu.make_async_copy(
                    table_hbm.at[idx_v.at[b, pl.ds(STARTS[j], n)]],
                    bufs[m],
                    gsems[m],
                ).wait()
                buf = bufs[m]
                fresh = (j == 0)

                @plsc.parallel_loop(0, D // NLANE, unroll=4)
                def _(d, n=n, buf=buf, fresh=fresh, arow=arow):
                    col = pl.ds(d * NLANE, NLANE)
                    vs = [buf[r, col] for r in range(n)]
                    while len(vs) > 1:
                        nxt = [vs[i] + vs[i + 1]
                               for i in range(0, len(vs) - 1, 2)]
                        if len(vs) % 2:
                            nxt.append(vs[-1])
                        vs = nxt
                    if fresh:
                        acc_v[arow, col] = vs[0]
                    else:
                        plsc.addupdate(acc_v.at[arow, col], vs[0])

                @pl.when(b < BAGS_PER_W - 1)
                def _(j=j):
                    fire(b + 1, j)

            pltpu.async_copy(acc_v.at[arow], out_hbm.at[base + b], osem)
            return carry

        lax.fori_loop(0, BAGS_PER_W, bag_body, 0, unroll=False)
        pltpu.make_async_copy(acc_v.at[0], out_hbm.at[base], osem).wait()
        pltpu.make_async_copy(acc_v.at[0], out_hbm.at[base], osem).wait()

    return pool(idx_pad, embedding)


def _tc_head(pooled, idx_pad, stm2d, e0, W1a, W1b, b1, W2, b2):
    BLK = 256
    grid = (B // BLK,)

    def body(sw_ref, sb_ref, iw_ref, ib_ref, stm_ref, e0_ref,
             w1a_ref, w1b_ref, b1_ref, w2_ref, b2_ref, out_ref):
        valid = lax.broadcasted_iota(jnp.int32, (BLK, LP), 1) < L
        cw = jnp.sum(jnp.where((iw_ref[...] == 0) & valid, 1.0, 0.0),
                     axis=1, keepdims=True)
        cb = jnp.sum(jnp.where((ib_ref[...] == 0) & valid, 1.0, 0.0),
                     axis=1, keepdims=True)
        e0 = e0_ref[...]
        w = sw_ref[...] - cw * e0
        bk = sb_ref[...] - cb * e0
        pick = stm_ref[...] == 1
        u = jnp.where(pick, w, bk)
        v = jnp.where(pick, bk, w)
        h = (jnp.dot(u, w1a_ref[...], preferred_element_type=jnp.float32)
             + jnp.dot(v, w1b_ref[...], preferred_element_type=jnp.float32)
             + b1_ref[...])
        h = jnp.maximum(h, 0.0)
        out_ref[...] = (jnp.dot(h, w2_ref[...],
                                preferred_element_type=jnp.float32)
                        + b2_ref[...])

    nblk = B // BLK
    return pl.pallas_call(
        body,
        grid=grid,
        in_specs=[
            pl.BlockSpec((BLK, D), lambda i: (i, 0)),
            pl.BlockSpec((BLK, D), lambda i, n=nblk: (i + n, 0)),
            pl.BlockSpec((BLK, LP), lambda i: (i, 0)),
            pl.BlockSpec((BLK, LP), lambda i, n=nblk: (i + n, 0)),
            pl.BlockSpec((BLK, 1), lambda i: (i, 0)),
            pl.BlockSpec((1, D), lambda i: (0, 0)),
            pl.BlockSpec((D, 128), lambda i: (0, 0)),
            pl.BlockSpec((D, 128), lambda i: (0, 0)),
            pl.BlockSpec((1, 128), lambda i: (0, 0)),
            pl.BlockSpec((128, 1), lambda i: (0, 0)),
            pl.BlockSpec((1, 1), lambda i: (0, 0)),
        ],
        out_specs=pl.BlockSpec((BLK, 1), lambda i: (i, 0)),
        out_shape=jax.ShapeDtypeStruct((B, 1), jnp.float32),
    )(pooled, pooled, idx_pad, idx_pad, stm2d, e0, W1a, W1b, b1, W2, b2)


def kernel(white, black, stm, embedding, W1, b1, W2, b2):
    idx = jnp.concatenate([white, black], axis=0).astype(jnp.int32)
    idx_pad = jnp.pad(idx, ((0, 0), (0, LP - L)))
    pooled = _sc_pool(idx_pad, embedding)
    stm2d = stm.astype(jnp.int32).reshape(B, 1)
    e0 = embedding[0:1]
    W1a = W1[:D]
    W1b = W1[D:]
    out = _tc_head(pooled, idx_pad, stm2d, e0, W1a, W1b,
                   b1.reshape(1, 128), W2, b2.reshape(1, 1))
    return out


# speedup vs baseline: 1.0030x; 1.0030x over previous
import functools

import jax
import jax.numpy as jnp
from jax import lax
from jax.experimental import pallas as pl
from jax.experimental.pallas import tpu as pltpu
from jax.experimental.pallas import tpu_sc as plsc

D = 2048
B = 1024
L = 50
LP = 56
NBAGS = 2 * B
NW = 32
BAGS_PER_W = NBAGS // NW
NLANE = 16
CHUNKS = (16, 16, 16, 2)
STARTS = (0, 16, 32, 48)


def _sc_pool(idx_pad, embedding):
    mesh = plsc.VectorSubcoreMesh(core_axis_name="c", subcore_axis_name="s")

    @functools.partial(
        pl.kernel,
        out_type=jax.ShapeDtypeStruct((NBAGS, D), jnp.float32),
        mesh=mesh,
        scratch_types=[
            pltpu.VMEM((BAGS_PER_W, LP), jnp.int32),
            pltpu.VMEM((16, D), jnp.float32),
            pltpu.VMEM((16, D), jnp.float32),
            pltpu.VMEM((16, D), jnp.float32),
            pltpu.VMEM((2, D), jnp.float32),
            pltpu.VMEM((2, D), jnp.float32),
            pltpu.SemaphoreType.DMA,
            pltpu.SemaphoreType.DMA,
            pltpu.SemaphoreType.DMA,
            pltpu.SemaphoreType.DMA,
            pltpu.SemaphoreType.DMA,
        ],
    )
    def pool(idx_hbm, table_hbm, out_hbm, idx_v, buf0, buf1, buf2, buf3,
             acc_v, gsem0, gsem1, gsem2, gsem3, osem):
        nc = plsc.get_sparse_core_info().num_cores
        wid = lax.axis_index("s") * nc + lax.axis_index("c")
        base = wid * BAGS_PER_W
        pltpu.sync_copy(idx_hbm.at[pl.ds(base, BAGS_PER_W)], idx_v)

        bufs = (buf0, buf1, buf2, buf3)
        gsems = (gsem0, gsem1, gsem2, gsem3)
        BUFMAP = (0, 1, 2, 3)

        def fire(bag, j):
            n = CHUNKS[j]
            m = BUFMAP[j]
            pltpu.async_copy(
                table_hbm.at[idx_v.at[bag, pl.ds(STARTS[j], n)]],
                bufs[m],
                gsems[m],
            )

        for j in range(len(CHUNKS)):
            fire(0, j)

        def bag_body(b, carry):
            arow = b & 1
            for j in range(len(CHUNKS)):
                n = CHUNKS[j]
                m = BUFMAP[j]
                if j == 0:
                    @pl.when(b >= 2)
                    def _():
                        pltpu.make_async_copy(
                            acc_v.at[0], out_hbm.at[base], osem).wait()
                pltpu.make_async_copy(
                    table_hbm.at[idx_v.at[b, pl.ds(STARTS[j], n)]],
                    bufs[m],
                    gsems[m],
                ).wait()
                buf = bufs[m]
                fresh = (j == 0)

                @plsc.parallel_loop(0, D // NLANE, unroll=8)
                def _(d, n=n, buf=buf, fresh=fresh, arow=arow):
                    col = pl.ds(d * NLANE, NLANE)
                    vs = [buf[r, col] for r in range(n)]
                    while len(vs) > 1:
                        nxt = [vs[i] + vs[i + 1]
                               for i in range(0, len(vs) - 1, 2)]
                        if len(vs) % 2:
                            nxt.append(vs[-1])
                        vs = nxt
                    if fresh:
                        acc_v[arow, col] = vs[0]
                    else:
                        plsc.addupdate(acc_v.at[arow, col], vs[0])

                @pl.when(b < BAGS_PER_W - 1)
                def _(j=j):
                    fire(b + 1, j)

            pltpu.async_copy(acc_v.at[arow], out_hbm.at[base + b], osem)
            return carry

        lax.fori_loop(0, BAGS_PER_W, bag_body, 0, unroll=False)
        pltpu.make_async_copy(acc_v.at[0], out_hbm.at[base], osem).wait()
        pltpu.make_async_copy(acc_v.at[0], out_hbm.at[base], osem).wait()

    return pool(idx_pad, embedding)


def _tc_head(pooled, idx_pad, stm2d, e0, W1a, W1b, b1, W2, b2):
    BLK = 256
    grid = (B // BLK,)

    def body(sw_ref, sb_ref, iw_ref, ib_ref, stm_ref, e0_ref,
             w1a_ref, w1b_ref, b1_ref, w2_ref, b2_ref, out_ref):
        valid = lax.broadcasted_iota(jnp.int32, (BLK, LP), 1) < L
        cw = jnp.sum(jnp.where((iw_ref[...] == 0) & valid, 1.0, 0.0),
                     axis=1, keepdims=True)
        cb = jnp.sum(jnp.where((ib_ref[...] == 0) & valid, 1.0, 0.0),
                     axis=1, keepdims=True)
        e0 = e0_ref[...]
        w = sw_ref[...] - cw * e0
        bk = sb_ref[...] - cb * e0
        pick = stm_ref[...] == 1
        u = jnp.where(pick, w, bk)
        v = jnp.where(pick, bk, w)
        h = (jnp.dot(u, w1a_ref[...], preferred_element_type=jnp.float32)
             + jnp.dot(v, w1b_ref[...], preferred_element_type=jnp.float32)
             + b1_ref[...])
        h = jnp.maximum(h, 0.0)
        out_ref[...] = (jnp.dot(h, w2_ref[...],
                                preferred_element_type=jnp.float32)
                        + b2_ref[...])

    nblk = B // BLK
    return pl.pallas_call(
        body,
        grid=grid,
        in_specs=[
            pl.BlockSpec((BLK, D), lambda i: (i, 0)),
            pl.BlockSpec((BLK, D), lambda i, n=nblk: (i + n, 0)),
            pl.BlockSpec((BLK, LP), lambda i: (i, 0)),
            pl.BlockSpec((BLK, LP), lambda i, n=nblk: (i + n, 0)),
            pl.BlockSpec((BLK, 1), lambda i: (i, 0)),
            pl.BlockSpec((1, D), lambda i: (0, 0)),
            pl.BlockSpec((D, 128), lambda i: (0, 0)),
            pl.BlockSpec((D, 128), lambda i: (0, 0)),
            pl.BlockSpec((1, 128), lambda i: (0, 0)),
            pl.BlockSpec((128, 1), lambda i: (0, 0)),
            pl.BlockSpec((1, 1), lambda i: (0, 0)),
        ],
        out_specs=pl.BlockSpec((BLK, 1), lambda i: (i, 0)),
        out_shape=jax.ShapeDtypeStruct((B, 1), jnp.float32),
    )(pooled, pooled, idx_pad, idx_pad, stm2d, e0, W1a, W1b, b1, W2, b2)


def kernel(white, black, stm, embedding, W1, b1, W2, b2):
    idx = jnp.concatenate([white, black], axis=0).astype(jnp.int32)
    idx_pad = jnp.pad(idx, ((0, 0), (0, LP - L)))
    pooled = _sc_pool(idx_pad, embedding)
    stm2d = stm.astype(jnp.int32).reshape(B, 1)
    e0 = embedding[0:1]
    W1a = W1[:D]
    W1b = W1[D:]
    out = _tc_head(pooled, idx_pad, stm2d, e0, W1a, W1b,
                   b1.reshape(1, 128), W2, b2.reshape(1, 1))
    return out
